# Initial kernel scaffold; baseline (speedup 1.0000x reference)
import jax
import jax.numpy as jnp
from jax.experimental import pallas as pl


def kernel(x, edge_index, batch, graph_stats, lin1_w, lin1_b, bn_g, bn_b, lin2_w, lin2_b, stat_w, stat_b, fc_w, fc_b):
    raise NotImplementedError("write your pallas kernel here")


# speedup vs baseline: 7.9437x; 7.9437x over previous
import functools
import math

import jax
import jax.numpy as jnp
from jax import lax
from jax.experimental import pallas as pl
from jax.experimental.pallas import tpu as pltpu
from jax.experimental.pallas import tpu_sc as plsc

_NS = 16
_W = 128
_C = 16
_LEAK = 0.2


def _leaky(v):
    return jnp.where(v >= 0, v, _LEAK * v)


def _dot(a, b):
    return jnp.dot(a, b, preferred_element_type=jnp.float32,
                   precision=lax.Precision.HIGHEST)


def _seg_sum_sc(h_pad, src_w, dst_w):
    np_, d = h_pad.shape
    nk = src_w.shape[1]
    t = nk * _C
    rps = np_ // _NS
    mesh = plsc.VectorSubcoreMesh(core_axis_name="c", subcore_axis_name="s",
                                  num_cores=1)

    @functools.partial(
        pl.kernel,
        out_type=jax.ShapeDtypeStruct((np_, d), jnp.float32),
        mesh=mesh,
        scratch_types=[
            pltpu.VMEM((2, _C, _W), jnp.int32),
            pltpu.VMEM((2, _C, _W), jnp.int32),
            pltpu.VMEM((_W, d), jnp.float32),
            pltpu.VMEM((_W, d), jnp.float32),
            pltpu.VMEM_SHARED((np_, d), jnp.float32),
            pltpu.SemaphoreType.DMA,
            pltpu.SemaphoreType.DMA,
        ],
    )
    def k(h_hbm, src_hbm, dst_hbm, out_hbm, src_v, dst_v, buf_a, buf_b,
          acc, gsem, isem):
        s = lax.axis_index("s")

        def wait_idx():
            pltpu.make_async_copy(src_hbm.at[s, 0], src_v.at[0], isem).wait()
            pltpu.make_async_copy(dst_hbm.at[s, 0], dst_v.at[0], isem).wait()

        pltpu.async_copy(src_hbm.at[s, 0], src_v.at[0], isem)
        pltpu.async_copy(dst_hbm.at[s, 0], dst_v.at[0], isem)
        pltpu.sync_copy(h_hbm.at[pl.ds(s * rps, rps)],
                        acc.at[pl.ds(s * rps, rps)])
        wait_idx()
        plsc.subcore_barrier()

        pltpu.async_copy(h_hbm.at[src_v.at[0, 0]], buf_a, gsem)

        @pl.loop(0, t, step=2)
        def _(w):
            k_ = w // _C
            kp = lax.rem(k_, 2)

            @pl.when(lax.rem(w, _C) == 0)
            def _():
                @pl.when(k_ + 1 < nk)
                def _():
                    knp = lax.rem(k_ + 1, 2)
                    pltpu.async_copy(src_hbm.at[s, k_ + 1], src_v.at[knp],
                                     isem)
                    pltpu.async_copy(dst_hbm.at[s, k_ + 1], dst_v.at[knp],
                                     isem)

            pltpu.make_async_copy(h_hbm.at[src_v.at[0, 0]], buf_a,
                                  gsem).wait()
            pltpu.async_copy(h_hbm.at[src_v.at[kp, lax.rem(w + 1, _C)]],
                             buf_b, gsem)
            pltpu.sync_copy(buf_a, acc.at[dst_v.at[kp, lax.rem(w, _C)]],
                            add=True)
            pltpu.make_async_copy(h_hbm.at[src_v.at[0, 0]], buf_b,
                                  gsem).wait()

            @pl.when(w + 2 < t)
            def _():
                @pl.when(lax.rem(w + 2, _C) == 0)
                def _():
                    wait_idx()

                pltpu.async_copy(
                    h_hbm.at[src_v.at[lax.rem((w + 2) // _C, 2),
                                      lax.rem(w + 2, _C)]],
                    buf_a, gsem)

            pltpu.sync_copy(buf_b, acc.at[dst_v.at[kp, lax.rem(w + 1, _C)]],
                            add=True)

        plsc.subcore_barrier()
        pltpu.sync_copy(acc.at[pl.ds(s * rps, rps)],
                        out_hbm.at[pl.ds(s * rps, rps)])

    return k(h_pad, src_w, dst_w)


def _proj_tc(batch3d, gs_pad, stat_w_pad, stat_b, np_, r):
    nb, _, _ = batch3d.shape
    nl, _, h_dim = stat_w_pad.shape
    b = gs_pad.shape[0]

    def body(b_ref, gs_ref, w_ref, bias_ref, out_ref):
        gs = jnp.nan_to_num(gs_ref[...], nan=-100.0)
        bvec = b_ref[0, 0, :]
        oh = (bvec[:, None]
              == lax.broadcasted_iota(jnp.int32, (r, b), 1)).astype(jnp.float32)
        for l in range(nl):
            g = _dot(gs, w_ref[l]) + bias_ref[l][None, :]
            out_ref[l] = _dot(oh, g)

    return pl.pallas_call(
        body,
        grid=(nb,),
        in_specs=[
            pl.BlockSpec((1, 1, r), lambda i: (i, 0, 0)),
            pl.BlockSpec(gs_pad.shape, lambda i: (0, 0)),
            pl.BlockSpec(stat_w_pad.shape, lambda i: (0, 0, 0)),
            pl.BlockSpec(stat_b.shape, lambda i: (0, 0)),
        ],
        out_specs=pl.BlockSpec((nl, r, h_dim), lambda i: (0, i, 0)),
        out_shape=jax.ShapeDtypeStruct((nl, np_, h_dim), jnp.float32),
    )(batch3d, gs_pad, stat_w_pad, stat_b)


def _layer_tc(hagg, proj_l, w1, b1, scale, shift, w2, b2, fcw, fcb, last, r):
    np_, h_dim = hagg.shape
    nb = np_ // r

    def body(p_ref, pr_ref, w1_ref, b1_ref, sc_ref, sh_ref, w2_ref, b2_ref,
             fw_ref, fb_ref, out_ref):
        z = _leaky(_dot(p_ref[...], w1_ref[...]) + b1_ref[...])
        z = z * sc_ref[...] + sh_ref[...]
        z = _leaky(_dot(z, w2_ref[...]) + b2_ref[...])
        z = z + pr_ref[...]
        if last:
            z = _dot(z, fw_ref[...]) + fb_ref[...]
        out_ref[...] = z

    return pl.pallas_call(
        body,
        grid=(nb,),
        in_specs=[
            pl.BlockSpec((r, h_dim), lambda i: (i, 0)),
            pl.BlockSpec((r, h_dim), lambda i: (i, 0)),
            pl.BlockSpec(w1.shape, lambda i: (0, 0)),
            pl.BlockSpec(b1.shape, lambda i: (0, 0)),
            pl.BlockSpec(scale.shape, lambda i: (0, 0)),
            pl.BlockSpec(shift.shape, lambda i: (0, 0)),
            pl.BlockSpec(w2.shape, lambda i: (0, 0)),
            pl.BlockSpec(b2.shape, lambda i: (0, 0)),
            pl.BlockSpec(fcw.shape, lambda i: (0, 0)),
            pl.BlockSpec(fcb.shape, lambda i: (0, 0)),
        ],
        out_specs=pl.BlockSpec((r, h_dim), lambda i: (i, 0)),
        out_shape=jax.ShapeDtypeStruct((np_, h_dim), jnp.float32),
    )(hagg, proj_l, w1, b1, scale, shift, w2, b2, fcw, fcb)


def kernel(x, edge_index, batch, graph_stats, lin1_w, lin1_b, bn_g, bn_b,
           lin2_w, lin2_b, stat_w, stat_b, fc_w, fc_b):
    n, d = x.shape
    nl, _, h_dim = lin1_w.shape
    e = edge_index.shape[1]

    np_ = (n // (_NS * 64) + 1) * (_NS * 64)

    nk = math.ceil(e / (_NS * _C * _W))
    t = nk * _C
    pad = _NS * t * _W - e
    src = edge_index[0]
    dst = edge_index[1]
    if pad:
        pidx = jnp.arange(pad, dtype=jnp.int32)
        src = jnp.concatenate([src, pidx % n])
        dst = jnp.concatenate([dst, n + pidx % (np_ - n)])
    src_w = src.reshape(_NS, nk, _C, _W)
    dst_w = dst.reshape(_NS, nk, _C, _W)

    h = jnp.pad(x, ((0, np_ - n), (0, 0)))

    r = 512
    batch3d = jnp.pad(batch, (0, np_ - n)).reshape(np_ // r, 1, r)
    gs_pad = jnp.pad(graph_stats, ((0, 0), (0, 1)))
    stat_w_pad = jnp.pad(stat_w, ((0, 0), (0, 1), (0, 0)))
    proj = _proj_tc(batch3d, gs_pad, stat_w_pad, stat_b, np_, r)

    inv_std = 1.0 / math.sqrt(1.0 + 1e-5)
    for i in range(nl):
        hagg = _seg_sum_sc(h, src_w, dst_w)
        h = _layer_tc(hagg, proj[i], lin1_w[i],
                      lin1_b[i][None, :], (bn_g[i] * inv_std)[None, :],
                      bn_b[i][None, :], lin2_w[i], lin2_b[i][None, :],
                      fc_w, fc_b[None, :], i == nl - 1, r)
    return h[:n]


# speedup vs baseline: 9.6644x; 1.2166x over previous
import functools
import math

import jax
import jax.numpy as jnp
from jax import lax
from jax.experimental import pallas as pl
from jax.experimental.pallas import tpu as pltpu
from jax.experimental.pallas import tpu_sc as plsc

_NS = 16
_W = 128
_C = 16
_LEAK = 0.2


def _leaky(v):
    return jnp.where(v >= 0, v, _LEAK * v)


def _dot(a, b):
    return jnp.dot(a, b, preferred_element_type=jnp.float32,
                   precision=lax.Precision.HIGHEST)


def _seg_sum_sc(h_pad, src_w, dst_w):
    np_, d = h_pad.shape
    nk = src_w.shape[1]
    t = nk * _C
    rps = np_ // _NS
    mesh = plsc.VectorSubcoreMesh(core_axis_name="c", subcore_axis_name="s",
                                  num_cores=1)

    @functools.partial(
        pl.kernel,
        out_type=jax.ShapeDtypeStruct((np_, d), jnp.float32),
        mesh=mesh,
        scratch_types=[
            pltpu.VMEM((2, _C, _W), jnp.int32),
            pltpu.VMEM((2, _C, _W), jnp.int32),
            pltpu.VMEM((_W, d), jnp.float32),
            pltpu.VMEM((_W, d), jnp.float32),
            pltpu.VMEM_SHARED((np_, d), jnp.float32),
            pltpu.SemaphoreType.DMA,
            pltpu.SemaphoreType.DMA,
            pltpu.SemaphoreType.DMA,
        ],
    )
    def k(h_hbm, src_hbm, dst_hbm, out_hbm, src_v, dst_v, buf_a, buf_b,
          acc, gsem_a, gsem_b, isem):
        s = lax.axis_index("s")

        def wait_idx():
            pltpu.make_async_copy(src_hbm.at[s, 0], src_v.at[0], isem).wait()
            pltpu.make_async_copy(dst_hbm.at[s, 0], dst_v.at[0], isem).wait()

        pltpu.async_copy(src_hbm.at[s, 0], src_v.at[0], isem)
        pltpu.async_copy(dst_hbm.at[s, 0], dst_v.at[0], isem)
        pltpu.sync_copy(h_hbm.at[pl.ds(s * rps, rps)],
                        acc.at[pl.ds(s * rps, rps)])
        wait_idx()
        plsc.subcore_barrier()

        pltpu.async_copy(h_hbm.at[src_v.at[0, 0]], buf_a, gsem_a)
        pltpu.async_copy(h_hbm.at[src_v.at[0, 1]], buf_b, gsem_b)

        @pl.loop(0, t, step=2)
        def _(w):
            k_ = w // _C
            kp = lax.rem(k_, 2)

            @pl.when(lax.rem(w, _C) == 0)
            def _():
                @pl.when(k_ + 1 < nk)
                def _():
                    knp = lax.rem(k_ + 1, 2)
                    pltpu.async_copy(src_hbm.at[s, k_ + 1], src_v.at[knp],
                                     isem)
                    pltpu.async_copy(dst_hbm.at[s, k_ + 1], dst_v.at[knp],
                                     isem)

            pltpu.make_async_copy(h_hbm.at[src_v.at[0, 0]], buf_a,
                                  gsem_a).wait()
            pltpu.sync_copy(buf_a, acc.at[dst_v.at[kp, lax.rem(w, _C)]],
                            add=True)

            @pl.when(w + 2 < t)
            def _():
                @pl.when(lax.rem(w + 2, _C) == 0)
                def _():
                    wait_idx()

                pltpu.async_copy(
                    h_hbm.at[src_v.at[lax.rem((w + 2) // _C, 2),
                                      lax.rem(w + 2, _C)]],
                    buf_a, gsem_a)

            pltpu.make_async_copy(h_hbm.at[src_v.at[0, 0]], buf_b,
                                  gsem_b).wait()
            pltpu.sync_copy(buf_b, acc.at[dst_v.at[kp, lax.rem(w + 1, _C)]],
                            add=True)

            @pl.when(w + 3 < t)
            def _():
                pltpu.async_copy(
                    h_hbm.at[src_v.at[lax.rem((w + 3) // _C, 2),
                                      lax.rem(w + 3, _C)]],
                    buf_b, gsem_b)

        plsc.subcore_barrier()
        pltpu.sync_copy(acc.at[pl.ds(s * rps, rps)],
                        out_hbm.at[pl.ds(s * rps, rps)])

    return k(h_pad, src_w, dst_w)


def _proj_tc(batch3d, gs_pad, stat_w_pad, stat_b, np_, r):
    nb, _, _ = batch3d.shape
    nl, _, h_dim = stat_w_pad.shape
    b = gs_pad.shape[0]

    def body(b_ref, gs_ref, w_ref, bias_ref, out_ref):
        gs = jnp.nan_to_num(gs_ref[...], nan=-100.0)
        bvec = b_ref[0, 0, :]
        oh = (bvec[:, None]
              == lax.broadcasted_iota(jnp.int32, (r, b), 1)).astype(jnp.float32)
        for l in range(nl):
            g = _dot(gs, w_ref[l]) + bias_ref[l][None, :]
            out_ref[l] = _dot(oh, g)

    return pl.pallas_call(
        body,
        grid=(nb,),
        in_specs=[
            pl.BlockSpec((1, 1, r), lambda i: (i, 0, 0)),
            pl.BlockSpec(gs_pad.shape, lambda i: (0, 0)),
            pl.BlockSpec(stat_w_pad.shape, lambda i: (0, 0, 0)),
            pl.BlockSpec(stat_b.shape, lambda i: (0, 0)),
        ],
        out_specs=pl.BlockSpec((nl, r, h_dim), lambda i: (0, i, 0)),
        out_shape=jax.ShapeDtypeStruct((nl, np_, h_dim), jnp.float32),
    )(batch3d, gs_pad, stat_w_pad, stat_b)


def _layer_tc(hagg, proj_l, w1, b1, scale, shift, w2, b2, fcw, fcb, last, r):
    np_, h_dim = hagg.shape
    nb = np_ // r

    def body(p_ref, pr_ref, w1_ref, b1_ref, sc_ref, sh_ref, w2_ref, b2_ref,
             fw_ref, fb_ref, out_ref):
        z = _leaky(_dot(p_ref[...], w1_ref[...]) + b1_ref[...])
        z = z * sc_ref[...] + sh_ref[...]
        z = _leaky(_dot(z, w2_ref[...]) + b2_ref[...])
        z = z + pr_ref[...]
        if last:
            z = _dot(z, fw_ref[...]) + fb_ref[...]
        out_ref[...] = z

    return pl.pallas_call(
        body,
        grid=(nb,),
        in_specs=[
            pl.BlockSpec((r, h_dim), lambda i: (i, 0)),
            pl.BlockSpec((r, h_dim), lambda i: (i, 0)),
            pl.BlockSpec(w1.shape, lambda i: (0, 0)),
            pl.BlockSpec(b1.shape, lambda i: (0, 0)),
            pl.BlockSpec(scale.shape, lambda i: (0, 0)),
            pl.BlockSpec(shift.shape, lambda i: (0, 0)),
            pl.BlockSpec(w2.shape, lambda i: (0, 0)),
            pl.BlockSpec(b2.shape, lambda i: (0, 0)),
            pl.BlockSpec(fcw.shape, lambda i: (0, 0)),
            pl.BlockSpec(fcb.shape, lambda i: (0, 0)),
        ],
        out_specs=pl.BlockSpec((r, h_dim), lambda i: (i, 0)),
        out_shape=jax.ShapeDtypeStruct((np_, h_dim), jnp.float32),
    )(hagg, proj_l, w1, b1, scale, shift, w2, b2, fcw, fcb)


def kernel(x, edge_index, batch, graph_stats, lin1_w, lin1_b, bn_g, bn_b,
           lin2_w, lin2_b, stat_w, stat_b, fc_w, fc_b):
    n, d = x.shape
    nl, _, h_dim = lin1_w.shape
    e = edge_index.shape[1]

    np_ = (n // (_NS * 64) + 1) * (_NS * 64)

    nk = math.ceil(e / (_NS * _C * _W))
    t = nk * _C
    pad = _NS * t * _W - e
    src = edge_index[0]
    dst = edge_index[1]
    if pad:
        pidx = jnp.arange(pad, dtype=jnp.int32)
        src = jnp.concatenate([src, pidx % n])
        dst = jnp.concatenate([dst, n + pidx % (np_ - n)])
    src_w = src.reshape(_NS, nk, _C, _W)
    dst_w = dst.reshape(_NS, nk, _C, _W)

    h = jnp.pad(x, ((0, np_ - n), (0, 0)))

    r = 512
    batch3d = jnp.pad(batch, (0, np_ - n)).reshape(np_ // r, 1, r)
    gs_pad = jnp.pad(graph_stats, ((0, 0), (0, 1)))
    stat_w_pad = jnp.pad(stat_w, ((0, 0), (0, 1), (0, 0)))
    proj = _proj_tc(batch3d, gs_pad, stat_w_pad, stat_b, np_, r)

    inv_std = 1.0 / math.sqrt(1.0 + 1e-5)
    for i in range(nl):
        hagg = _seg_sum_sc(h, src_w, dst_w)
        h = _layer_tc(hagg, proj[i], lin1_w[i],
                      lin1_b[i][None, :], (bn_g[i] * inv_std)[None, :],
                      bn_b[i][None, :], lin2_w[i], lin2_b[i][None, :],
                      fc_w, fc_b[None, :], i == nl - 1, r)
    return h[:n]
